# Initial kernel scaffold; baseline (speedup 1.0000x reference)
import functools

import jax
import jax.numpy as jnp
from jax import lax
from jax.experimental import pallas as pl
from jax.experimental.pallas import tpu as pltpu
from jax.experimental.pallas import tpu_sc as plsc

N = 10000
D = 128
H = 128
B = 1024
E = 320000

ROWS_PER_TILE = 160
EPAD = 16 * ROWS_PER_TILE * 128
NGROUP = ROWS_PER_TILE // 8

NACC = 16 * 626
NDEG = 16 * 640

_BR = 500


def _stats_body(x_ref, o_ref):
    i = pl.program_id(0)
    xb = x_ref[...]
    s = jnp.sum(xb, axis=0, keepdims=True)
    s2 = jnp.sum(xb * xb, axis=0, keepdims=True)

    @pl.when(i == 0)
    def _():
        o_ref[...] = jnp.zeros_like(o_ref)

    o_ref[0:1, :] += s
    o_ref[1:2, :] += s2

    @pl.when(i == pl.num_programs(0) - 1)
    def _():
        mean = o_ref[0:1, :] / N
        ms = o_ref[1:2, :] / N
        var = ms - mean * mean
        o_ref[0:1, :] = mean
        o_ref[1:2, :] = lax.rsqrt(var + 1e-5)


def _stats(x):
    return pl.pallas_call(
        _stats_body,
        grid=(N // _BR,),
        in_specs=[pl.BlockSpec((_BR, D), lambda i: (i, 0))],
        out_specs=pl.BlockSpec((8, D), lambda i: (0, 0)),
        out_shape=jax.ShapeDtypeStruct((8, D), jnp.float32),
    )(x)


def _mm1_body(x_ref, st_ref, g_ref, bta_ref, w_ref, up_ref, dn_ref, bs_ref):
    mean = st_ref[0:1, :]
    rstd = st_ref[1:2, :]
    xn = (x_ref[...] - mean) * (rstd * g_ref[...]) + bta_ref[...]
    y = jnp.dot(xn, w_ref[...], preferred_element_type=jnp.float32)
    up_ref[...] = y[:, :H]
    dn_ref[...] = y[:, H:2 * H]
    bs_ref[...] = y[:, 2 * H:]


def _mm1(x, stats, gamma, beta, w1cat):
    o = jax.ShapeDtypeStruct((N, H), jnp.float32)
    return pl.pallas_call(
        _mm1_body,
        grid=(N // _BR,),
        in_specs=[
            pl.BlockSpec((_BR, D), lambda i: (i, 0)),
            pl.BlockSpec((8, D), lambda i: (0, 0)),
            pl.BlockSpec((1, D), lambda i: (0, 0)),
            pl.BlockSpec((1, D), lambda i: (0, 0)),
            pl.BlockSpec((D, 3 * H), lambda i: (0, 0)),
        ],
        out_specs=[pl.BlockSpec((_BR, H), lambda i: (i, 0))] * 3,
        out_shape=[o, o, o],
    )(x, stats, gamma, beta, w1cat)


def _norm_cat(su_ref, sd_ref, bs_ref, du_ref, dv_ref):
    iu = 1.0 / jnp.maximum(du_ref[...], 1.0)
    iv = 1.0 / jnp.maximum(dv_ref[...], 1.0)
    cat = jnp.concatenate([su_ref[...] * iu, sd_ref[...] * iv, bs_ref[...]],
                          axis=1)
    ss = jnp.sum(cat * cat, axis=1, keepdims=True)
    l2 = jnp.maximum(jnp.sqrt(ss), 1e-12)
    cat = cat / l2
    return jnp.where(cat >= 0, cat, 0.1 * cat)


def _fused_body(su_ref, sd_ref, bs_ref, du_ref, dv_ref, w_ref,
                up_ref, dn_ref, bo_ref):
    h = _norm_cat(su_ref, sd_ref, bs_ref, du_ref, dv_ref)
    y = jnp.dot(h, w_ref[...], preferred_element_type=jnp.float32)
    up_ref[...] = y[:, :H]
    dn_ref[...] = y[:, H:2 * H]
    bo_ref[...] = y[:, 2 * H:]


def _fused(s_up, s_dn, bias, du, dv, wcat):
    o = jax.ShapeDtypeStruct((N, H), jnp.float32)
    blk = pl.BlockSpec((_BR, H), lambda i: (i, 0))
    dblk = pl.BlockSpec((_BR, 1), lambda i: (i, 0))
    return pl.pallas_call(
        _fused_body,
        grid=(N // _BR,),
        in_specs=[blk, blk, blk, dblk, dblk,
                  pl.BlockSpec((3 * H, 3 * H), lambda i: (0, 0))],
        out_specs=[blk] * 3,
        out_shape=[o, o, o],
    )(s_up, s_dn, bias, du, dv, wcat)


def _post3_body(su_ref, sd_ref, bs_ref, du_ref, dv_ref, h_ref):
    h_ref[...] = _norm_cat(su_ref, sd_ref, bs_ref, du_ref, dv_ref)


def _post3(s_up, s_dn, bias, du, dv):
    blk = pl.BlockSpec((_BR, H), lambda i: (i, 0))
    dblk = pl.BlockSpec((_BR, 1), lambda i: (i, 0))
    return pl.pallas_call(
        _post3_body,
        grid=(N // _BR,),
        in_specs=[blk, blk, blk, dblk, dblk],
        out_specs=pl.BlockSpec((_BR, 3 * H), lambda i: (i, 0)),
        out_shape=jax.ShapeDtypeStruct((N, 3 * H), jnp.float32),
    )(s_up, s_dn, bias, du, dv)


def _decode_body(a_ref, b_ref, p1_ref, p2_ref, o_ref):
    u = jnp.dot(a_ref[...], p1_ref[...], preferred_element_type=jnp.float32)
    v = jnp.dot(u, p2_ref[...], preferred_element_type=jnp.float32)
    w = jnp.dot(b_ref[...], p1_ref[...], preferred_element_type=jnp.float32)
    o_ref[...] = jnp.sum(v * w, axis=1, keepdims=True)


def _decode(a, b, p1, p2):
    bb = 256
    return pl.pallas_call(
        _decode_body,
        grid=(B // bb,),
        in_specs=[
            pl.BlockSpec((bb, 3 * H), lambda i: (i, 0)),
            pl.BlockSpec((bb, 3 * H), lambda i: (i, 0)),
            pl.BlockSpec((3 * H, 128), lambda i: (0, 0)),
            pl.BlockSpec((128, 128), lambda i: (0, 0)),
        ],
        out_specs=pl.BlockSpec((bb, 1), lambda i: (i, 0)),
        out_shape=jax.ShapeDtypeStruct((B, 1), jnp.float32),
    )(a, b, p1, p2)


_MESH = plsc.VectorSubcoreMesh(core_axis_name="c", subcore_axis_name="s")


def _agg_kernel_body(tu, td, su, du, sv, dv, ou, od,
                     acc, sidx, didx, rows, stage, sem):
    cid = lax.axis_index("c")
    sid = lax.axis_index("s")

    zv = jnp.zeros((16,), jnp.float32)

    def zrow(r, carry):
        for k in range(8):
            stage[r, pl.ds(k * 16, 16)] = zv
        return carry

    lax.fori_loop(0, 313, zrow, 0)
    off = sid * 626
    pltpu.sync_copy(stage, acc.at[pl.ds(off, 313)])
    pltpu.sync_copy(stage, acc.at[pl.ds(off + 313, 313)])
    plsc.subcore_barrier()

    def run_dir(tab, src2d, dst2d):
        base_row = sid * ROWS_PER_TILE

        def grp(g, carry):
            r0 = base_row + g * 8
            pltpu.sync_copy(src2d.at[pl.ds(r0, 8)], sidx)
            pltpu.sync_copy(dst2d.at[pl.ds(r0, 8)], didx)
            for j in range(8):
                pltpu.async_copy(tab.at[sidx.at[j]], rows, sem).wait()
                pltpu.sync_copy(rows, acc.at[didx.at[j]], add=True)
            return carry

        lax.fori_loop(0, NGROUP, grp, 0)

    @pl.when(cid == 0)
    def _():
        run_dir(tu, su, du)

    @pl.when(cid == 1)
    def _():
        run_dir(td, sv, dv)

    plsc.subcore_barrier()

    def wb(out_ref):
        @pl.when(sid < 15)
        def _():
            pltpu.sync_copy(acc.at[pl.ds(off, 313)], stage)
            pltpu.sync_copy(stage, out_ref.at[pl.ds(off, 313)])
            pltpu.sync_copy(acc.at[pl.ds(off + 313, 313)], stage)
            pltpu.sync_copy(stage, out_ref.at[pl.ds(off + 313, 313)])

        @pl.when(sid == 15)
        def _():
            pltpu.sync_copy(acc.at[pl.ds(off, 313)], stage)
            pltpu.sync_copy(stage, out_ref.at[pl.ds(off, 313)])
            pltpu.sync_copy(acc.at[pl.ds(off + 313, 297)],
                            stage.at[pl.ds(0, 297)])
            pltpu.sync_copy(stage.at[pl.ds(0, 297)],
                            out_ref.at[pl.ds(off + 313, 297)])

    @pl.when(cid == 0)
    def _():
        wb(ou)

    @pl.when(cid == 1)
    def _():
        wb(od)


def _agg(tab_up, tab_dn, src_up, dst_up, src_dn, dst_dn):
    o = jax.ShapeDtypeStruct((N, H), jnp.float32)
    f = pl.kernel(
        _agg_kernel_body,
        out_type=[o, o],
        mesh=_MESH,
        scratch_types=[
            pltpu.VMEM_SHARED((NACC, H), jnp.float32),
            pltpu.VMEM((8, 128), jnp.int32),
            pltpu.VMEM((8, 128), jnp.int32),
            pltpu.VMEM((128, H), jnp.float32),
            pltpu.VMEM((313, H), jnp.float32),
            pltpu.SemaphoreType.DMA,
        ],
    )
    return f(tab_up, tab_dn, src_up, dst_up, src_dn, dst_dn)


def _deg_kernel_body(du2d, dv2d, ou, od, dacc, didx, ones, zstage, sem):
    del sem
    cid = lax.axis_index("c")
    sid = lax.axis_index("s")

    ov = jnp.full((16,), 1.0, jnp.float32)
    for k in range(8):
        ones[pl.ds(k * 16, 16)] = ov
    zv = jnp.zeros((16,), jnp.float32)

    def zrow(r, carry):
        zstage[pl.ds(r * 16, 16)] = zv
        return carry

    lax.fori_loop(0, 40, zrow, 0)
    off = sid * 640
    pltpu.sync_copy(zstage, dacc.at[pl.ds(off, 640)])
    plsc.subcore_barrier()

    def run_dir(dst2d):
        base_row = sid * ROWS_PER_TILE

        def grp(g, carry):
            r0 = base_row + g * 8
            pltpu.sync_copy(dst2d.at[pl.ds(r0, 8)], didx)
            for j in range(8):
                pltpu.sync_copy(ones, dacc.at[didx.at[j]], add=True)
            return carry

        lax.fori_loop(0, NGROUP, grp, 0)

    @pl.when(cid == 0)
    def _():
        run_dir(du2d)

    @pl.when(cid == 1)
    def _():
        run_dir(dv2d)

    plsc.subcore_barrier()

    def wb(out_ref):
        @pl.when(sid < 15)
        def _():
            pltpu.sync_copy(dacc.at[pl.ds(off, 640)], zstage)
            pltpu.sync_copy(zstage, out_ref.at[pl.ds(off, 640)])

        @pl.when(sid == 15)
        def _():
            pltpu.sync_copy(dacc.at[pl.ds(off, 400)], zstage.at[pl.ds(0, 400)])
            pltpu.sync_copy(zstage.at[pl.ds(0, 400)],
                            out_ref.at[pl.ds(off, 400)])

    @pl.when(cid == 0)
    def _():
        wb(ou)

    @pl.when(cid == 1)
    def _():
        wb(od)


def _deg(dst_up, dst_dn):
    o = jax.ShapeDtypeStruct((N,), jnp.float32)
    f = pl.kernel(
        _deg_kernel_body,
        out_type=[o, o],
        mesh=_MESH,
        scratch_types=[
            pltpu.VMEM_SHARED((NDEG,), jnp.float32),
            pltpu.VMEM((8, 128), jnp.int32),
            pltpu.VMEM((128,), jnp.float32),
            pltpu.VMEM((640,), jnp.float32),
            pltpu.SemaphoreType.DMA,
        ],
    )
    return f(dst_up, dst_dn)


def _pair_kernel_body(h3, ia, ib, oa, ob, idxb, buf, sem):
    cid = lax.axis_index("c")
    sid = lax.axis_index("s")
    wid = sid * 2 + cid
    base = wid * 32

    pltpu.sync_copy(ia.at[pl.ds(base, 32)], idxb)
    pltpu.async_copy(h3.at[idxb], buf, sem).wait()
    pltpu.sync_copy(buf, oa.at[pl.ds(base, 32)])

    pltpu.sync_copy(ib.at[pl.ds(base, 32)], idxb)
    pltpu.async_copy(h3.at[idxb], buf, sem).wait()
    pltpu.sync_copy(buf, ob.at[pl.ds(base, 32)])


def _pair(h3, ia, ib):
    o = jax.ShapeDtypeStruct((B, 3 * H), jnp.float32)
    f = pl.kernel(
        _pair_kernel_body,
        out_type=[o, o],
        mesh=_MESH,
        scratch_types=[
            pltpu.VMEM((32,), jnp.int32),
            pltpu.VMEM((32, 3 * H), jnp.float32),
            pltpu.SemaphoreType.DMA,
        ],
    )
    return f(h3, ia, ib)


def kernel(x, edge_index, drug_index, bn_gamma, bn_beta,
           W_up1, W_down1, W_bias1,
           W_up2, W_down2, W_bias2,
           W_up3, W_down3, W_bias3,
           P1, P2):
    row = edge_index[0]
    col = edge_index[1]
    npad = EPAD - E
    pad_g = jnp.zeros((npad,), jnp.int32)
    pad_s = jnp.full((npad,), N, jnp.int32)
    shape2d = (EPAD // 128, 128)
    src_up = jnp.concatenate([row, pad_g]).reshape(shape2d)
    dst_up = jnp.concatenate([col, pad_s]).reshape(shape2d)
    src_dn = jnp.concatenate([col, pad_g]).reshape(shape2d)
    dst_dn = jnp.concatenate([row, pad_s]).reshape(shape2d)

    stats = _stats(x)
    w1cat = jnp.concatenate([W_up1, W_down1, W_bias1], axis=1)
    up1, dn1, b1 = _mm1(x, stats, bn_gamma.reshape(1, D),
                        bn_beta.reshape(1, D), w1cat)

    du, dv = _deg(dst_up, dst_dn)
    du = du.reshape(N, 1)
    dv = dv.reshape(N, 1)

    s_up, s_dn = _agg(up1, dn1, src_up, dst_up, src_dn, dst_dn)

    w2cat = jnp.concatenate([W_up2, W_down2, W_bias2], axis=1)
    up2, dn2, b2 = _fused(s_up, s_dn, b1, du, dv, w2cat)
    s_up, s_dn = _agg(up2, dn2, src_up, dst_up, src_dn, dst_dn)

    w3cat = jnp.concatenate([W_up3, W_down3, W_bias3], axis=1)
    up3, dn3, b3 = _fused(s_up, s_dn, b2, du, dv, w3cat)
    s_up, s_dn = _agg(up3, dn3, src_up, dst_up, src_dn, dst_dn)

    h3 = _post3(s_up, s_dn, b3, du, dv)

    ia = (drug_index[:, 0] - 1).astype(jnp.int32)
    ib = (drug_index[:, 1] - 1).astype(jnp.int32)
    a, b = _pair(h3, ia, ib)
    return _decode(a, b, P1, P2)


# speedup vs baseline: 4.9199x; 4.9199x over previous
import functools

import jax
import jax.numpy as jnp
from jax import lax
from jax.experimental import pallas as pl
from jax.experimental.pallas import tpu as pltpu
from jax.experimental.pallas import tpu_sc as plsc

N = 10000
D = 128
H = 128
B = 1024
E = 320000

ROWS_PER_TILE = 160
EPAD = 16 * ROWS_PER_TILE * 128
NGROUP = ROWS_PER_TILE // 8

NACC = 16 * 632
NDEG = 16 * 640

_BR = 1000


def _stats_body(x_ref, o_ref):
    i = pl.program_id(0)
    xb = x_ref[...]
    s = jnp.sum(xb, axis=0, keepdims=True)
    s2 = jnp.sum(xb * xb, axis=0, keepdims=True)

    @pl.when(i == 0)
    def _():
        o_ref[...] = jnp.zeros_like(o_ref)

    o_ref[0:1, :] += s
    o_ref[1:2, :] += s2

    @pl.when(i == pl.num_programs(0) - 1)
    def _():
        mean = o_ref[0:1, :] / N
        ms = o_ref[1:2, :] / N
        var = ms - mean * mean
        o_ref[0:1, :] = mean
        o_ref[1:2, :] = lax.rsqrt(var + 1e-5)


def _stats(x):
    return pl.pallas_call(
        _stats_body,
        grid=(N // _BR,),
        in_specs=[pl.BlockSpec((_BR, D), lambda i: (i, 0))],
        out_specs=pl.BlockSpec((8, D), lambda i: (0, 0)),
        out_shape=jax.ShapeDtypeStruct((8, D), jnp.float32),
    )(x)


def _mm1_body(x_ref, st_ref, g_ref, bta_ref, w_ref, up_ref, dn_ref, bs_ref):
    mean = st_ref[0:1, :]
    rstd = st_ref[1:2, :]
    xn = (x_ref[...] - mean) * (rstd * g_ref[...]) + bta_ref[...]
    y = jnp.dot(xn, w_ref[...], preferred_element_type=jnp.float32)
    up_ref[...] = y[:, :H]
    dn_ref[...] = y[:, H:2 * H]
    bs_ref[...] = y[:, 2 * H:]


def _mm1(x, stats, gamma, beta, w1cat):
    o = jax.ShapeDtypeStruct((N, H), jnp.float32)
    return pl.pallas_call(
        _mm1_body,
        grid=(N // _BR,),
        in_specs=[
            pl.BlockSpec((_BR, D), lambda i: (i, 0)),
            pl.BlockSpec((8, D), lambda i: (0, 0)),
            pl.BlockSpec((1, D), lambda i: (0, 0)),
            pl.BlockSpec((1, D), lambda i: (0, 0)),
            pl.BlockSpec((D, 3 * H), lambda i: (0, 0)),
        ],
        out_specs=[pl.BlockSpec((_BR, H), lambda i: (i, 0))] * 3,
        out_shape=[o, o, o],
    )(x, stats, gamma, beta, w1cat)


def _norm_cat(su_ref, sd_ref, bs_ref, du_ref, dv_ref):
    iu = 1.0 / jnp.maximum(du_ref[...], 1.0)
    iv = 1.0 / jnp.maximum(dv_ref[...], 1.0)
    cat = jnp.concatenate([su_ref[...] * iu, sd_ref[...] * iv, bs_ref[...]],
                          axis=1)
    ss = jnp.sum(cat * cat, axis=1, keepdims=True)
    l2 = jnp.maximum(jnp.sqrt(ss), 1e-12)
    cat = cat / l2
    return jnp.where(cat >= 0, cat, 0.1 * cat)


def _fused_body(su_ref, sd_ref, bs_ref, du_ref, dv_ref, w_ref,
                up_ref, dn_ref, bo_ref):
    h = _norm_cat(su_ref, sd_ref, bs_ref, du_ref, dv_ref)
    y = jnp.dot(h, w_ref[...], preferred_element_type=jnp.float32)
    up_ref[...] = y[:, :H]
    dn_ref[...] = y[:, H:2 * H]
    bo_ref[...] = y[:, 2 * H:]


def _fused(s_up, s_dn, bias, du, dv, wcat):
    o = jax.ShapeDtypeStruct((N, H), jnp.float32)
    blk = pl.BlockSpec((_BR, H), lambda i: (i, 0))
    dblk = pl.BlockSpec((_BR, 1), lambda i: (i, 0))
    return pl.pallas_call(
        _fused_body,
        grid=(N // _BR,),
        in_specs=[blk, blk, blk, dblk, dblk,
                  pl.BlockSpec((3 * H, 3 * H), lambda i: (0, 0))],
        out_specs=[blk] * 3,
        out_shape=[o, o, o],
    )(s_up, s_dn, bias, du, dv, wcat)


def _post3_body(su_ref, sd_ref, bs_ref, du_ref, dv_ref, h_ref):
    h_ref[...] = _norm_cat(su_ref, sd_ref, bs_ref, du_ref, dv_ref)


def _post3(s_up, s_dn, bias, du, dv):
    blk = pl.BlockSpec((_BR, H), lambda i: (i, 0))
    dblk = pl.BlockSpec((_BR, 1), lambda i: (i, 0))
    return pl.pallas_call(
        _post3_body,
        grid=(N // _BR,),
        in_specs=[blk, blk, blk, dblk, dblk],
        out_specs=pl.BlockSpec((_BR, 3 * H), lambda i: (i, 0)),
        out_shape=jax.ShapeDtypeStruct((N, 3 * H), jnp.float32),
    )(s_up, s_dn, bias, du, dv)


def _decode_body(a_ref, b_ref, p1_ref, p2_ref, o_ref):
    u = jnp.dot(a_ref[...], p1_ref[...], preferred_element_type=jnp.float32)
    v = jnp.dot(u, p2_ref[...], preferred_element_type=jnp.float32)
    w = jnp.dot(b_ref[...], p1_ref[...], preferred_element_type=jnp.float32)
    o_ref[...] = jnp.sum(v * w, axis=1, keepdims=True)


def _decode(a, b, p1, p2):
    bb = 256
    return pl.pallas_call(
        _decode_body,
        grid=(B // bb,),
        in_specs=[
            pl.BlockSpec((bb, 3 * H), lambda i: (i, 0)),
            pl.BlockSpec((bb, 3 * H), lambda i: (i, 0)),
            pl.BlockSpec((3 * H, 128), lambda i: (0, 0)),
            pl.BlockSpec((128, 128), lambda i: (0, 0)),
        ],
        out_specs=pl.BlockSpec((bb, 1), lambda i: (i, 0)),
        out_shape=jax.ShapeDtypeStruct((B, 1), jnp.float32),
    )(a, b, p1, p2)


_MESH = plsc.VectorSubcoreMesh(core_axis_name="c", subcore_axis_name="s")


def _agg_kernel_body(tu, td, su, du, sv, dv, ou, od,
                     acc, sidx, didx, rows, sem):
    cid = lax.axis_index("c")
    sid = lax.axis_index("s")

    zv = jnp.zeros((16,), jnp.float32)

    def zrow(r, carry):
        for k in range(8):
            rows[r, pl.ds(k * 16, 16)] = zv
        return carry

    lax.fori_loop(0, 128, zrow, 0)
    off = sid * 632
    for c in range(4):
        pltpu.sync_copy(rows, acc.at[pl.ds(off + c * 128, 128)])
    pltpu.sync_copy(rows.at[pl.ds(0, 120)], acc.at[pl.ds(off + 512, 120)])
    plsc.subcore_barrier()

    def run_dir(tab, src2d, dst2d):
        base_row = sid * ROWS_PER_TILE

        def grp(g, carry):
            r0 = base_row + g * 8
            pltpu.sync_copy(src2d.at[pl.ds(r0, 8)], sidx)
            pltpu.sync_copy(dst2d.at[pl.ds(r0, 8)], didx)
            for j in range(8):
                pltpu.async_copy(tab.at[sidx.at[j]], rows, sem).wait()
                pltpu.sync_copy(rows, acc.at[didx.at[j]], add=True)
            return carry

        lax.fori_loop(0, NGROUP, grp, 0)

    @pl.when(cid == 0)
    def _():
        run_dir(tu, su, du)

    @pl.when(cid == 1)
    def _():
        run_dir(td, sv, dv)

    plsc.subcore_barrier()

    def wb(out_ref):
        def chunk(o2, nrow):
            pltpu.sync_copy(acc.at[pl.ds(off + o2, nrow)],
                            rows.at[pl.ds(0, nrow)])
            pltpu.sync_copy(rows.at[pl.ds(0, nrow)],
                            out_ref.at[pl.ds(off + o2, nrow)])

        @pl.when(sid < 15)
        def _():
            for c in range(4):
                chunk(c * 128, 128)
            chunk(512, 120)

        @pl.when(sid == 15)
        def _():
            for c in range(4):
                chunk(c * 128, 128)
            chunk(512, 8)

    @pl.when(cid == 0)
    def _():
        wb(ou)

    @pl.when(cid == 1)
    def _():
        wb(od)


def _agg(tab_up, tab_dn, src_up, dst_up, src_dn, dst_dn):
    o = jax.ShapeDtypeStruct((N, H), jnp.float32)
    f = pl.kernel(
        _agg_kernel_body,
        out_type=[o, o],
        mesh=_MESH,
        scratch_types=[
            pltpu.VMEM_SHARED((NACC, H), jnp.float32),
            pltpu.VMEM((8, 128), jnp.int32),
            pltpu.VMEM((8, 128), jnp.int32),
            pltpu.VMEM((128, H), jnp.float32),
            pltpu.SemaphoreType.DMA,
        ],
    )
    return f(tab_up, tab_dn, src_up, dst_up, src_dn, dst_dn)


def _deg_kernel_body(du2d, dv2d, ou, od, dacc, didx, ones, zstage, sem):
    del sem
    cid = lax.axis_index("c")
    sid = lax.axis_index("s")

    ov = jnp.full((16,), 1.0, jnp.float32)
    for k in range(8):
        ones[pl.ds(k * 16, 16)] = ov
    zv = jnp.zeros((16,), jnp.float32)

    def zrow(r, carry):
        zstage[pl.ds(r * 16, 16)] = zv
        return carry

    lax.fori_loop(0, 40, zrow, 0)
    off = sid * 640
    pltpu.sync_copy(zstage, dacc.at[pl.ds(off, 640)])
    plsc.subcore_barrier()

    def run_dir(dst2d):
        base_row = sid * ROWS_PER_TILE

        def grp(g, carry):
            r0 = base_row + g * 8
            pltpu.sync_copy(dst2d.at[pl.ds(r0, 8)], didx)
            for j in range(8):
                pltpu.sync_copy(ones, dacc.at[didx.at[j]], add=True)
            return carry

        lax.fori_loop(0, NGROUP, grp, 0)

    @pl.when(cid == 0)
    def _():
        run_dir(du2d)

    @pl.when(cid == 1)
    def _():
        run_dir(dv2d)

    plsc.subcore_barrier()

    def wb(out_ref):
        @pl.when(sid < 15)
        def _():
            pltpu.sync_copy(dacc.at[pl.ds(off, 640)], zstage)
            pltpu.sync_copy(zstage, out_ref.at[pl.ds(off, 640)])

        @pl.when(sid == 15)
        def _():
            pltpu.sync_copy(dacc.at[pl.ds(off, 400)], zstage.at[pl.ds(0, 400)])
            pltpu.sync_copy(zstage.at[pl.ds(0, 400)],
                            out_ref.at[pl.ds(off, 400)])

    @pl.when(cid == 0)
    def _():
        wb(ou)

    @pl.when(cid == 1)
    def _():
        wb(od)


def _deg(dst_up, dst_dn):
    o = jax.ShapeDtypeStruct((N,), jnp.float32)
    f = pl.kernel(
        _deg_kernel_body,
        out_type=[o, o],
        mesh=_MESH,
        scratch_types=[
            pltpu.VMEM_SHARED((NDEG,), jnp.float32),
            pltpu.VMEM((8, 128), jnp.int32),
            pltpu.VMEM((128,), jnp.float32),
            pltpu.VMEM((640,), jnp.float32),
            pltpu.SemaphoreType.DMA,
        ],
    )
    return f(dst_up, dst_dn)


def _pair_kernel_body(h3, ia, ib, oa, ob, idxb, buf, sem):
    cid = lax.axis_index("c")
    sid = lax.axis_index("s")
    wid = sid * 2 + cid
    base = wid * 32

    pltpu.sync_copy(ia.at[pl.ds(base, 32)], idxb)
    pltpu.async_copy(h3.at[idxb], buf, sem).wait()
    pltpu.sync_copy(buf, oa.at[pl.ds(base, 32)])

    pltpu.sync_copy(ib.at[pl.ds(base, 32)], idxb)
    pltpu.async_copy(h3.at[idxb], buf, sem).wait()
    pltpu.sync_copy(buf, ob.at[pl.ds(base, 32)])


def _pair(h3, ia, ib):
    o = jax.ShapeDtypeStruct((B, 3 * H), jnp.float32)
    f = pl.kernel(
        _pair_kernel_body,
        out_type=[o, o],
        mesh=_MESH,
        scratch_types=[
            pltpu.VMEM((32,), jnp.int32),
            pltpu.VMEM((32, 3 * H), jnp.float32),
            pltpu.SemaphoreType.DMA,
        ],
    )
    return f(h3, ia, ib)


def kernel(x, edge_index, drug_index, bn_gamma, bn_beta,
           W_up1, W_down1, W_bias1,
           W_up2, W_down2, W_bias2,
           W_up3, W_down3, W_bias3,
           P1, P2):
    row = edge_index[0]
    col = edge_index[1]
    npad = EPAD - E
    pad_g = jnp.zeros((npad,), jnp.int32)
    pad_s = jnp.full((npad,), N, jnp.int32)
    shape2d = (EPAD // 128, 128)
    src_up = jnp.concatenate([row, pad_g]).reshape(shape2d)
    dst_up = jnp.concatenate([col, pad_s]).reshape(shape2d)
    src_dn = jnp.concatenate([col, pad_g]).reshape(shape2d)
    dst_dn = jnp.concatenate([row, pad_s]).reshape(shape2d)

    stats = _stats(x)
    w1cat = jnp.concatenate([W_up1, W_down1, W_bias1], axis=1)
    up1, dn1, b1 = _mm1(x, stats, bn_gamma.reshape(1, D),
                        bn_beta.reshape(1, D), w1cat)

    du, dv = _deg(dst_up, dst_dn)
    du = du.reshape(N, 1)
    dv = dv.reshape(N, 1)

    s_up, s_dn = _agg(up1, dn1, src_up, dst_up, src_dn, dst_dn)

    w2cat = jnp.concatenate([W_up2, W_down2, W_bias2], axis=1)
    up2, dn2, b2 = _fused(s_up, s_dn, b1, du, dv, w2cat)
    s_up, s_dn = _agg(up2, dn2, src_up, dst_up, src_dn, dst_dn)

    w3cat = jnp.concatenate([W_up3, W_down3, W_bias3], axis=1)
    up3, dn3, b3 = _fused(s_up, s_dn, b2, du, dv, w3cat)
    s_up, s_dn = _agg(up3, dn3, src_up, dst_up, src_dn, dst_dn)

    h3 = _post3(s_up, s_dn, b3, du, dv)

    ia = (drug_index[:, 0] - 1).astype(jnp.int32)
    ib = (drug_index[:, 1] - 1).astype(jnp.int32)
    a, b = _pair(h3, ia, ib)
    return _decode(a, b, P1, P2)


# speedup vs baseline: 5.6248x; 1.1433x over previous
import functools

import jax
import jax.numpy as jnp
from jax import lax
from jax.experimental import pallas as pl
from jax.experimental.pallas import tpu as pltpu
from jax.experimental.pallas import tpu_sc as plsc

N = 10000
D = 128
H = 128
B = 1024
E = 320000

ROWS_PER_TILE = 160
EPAD = 16 * ROWS_PER_TILE * 128
NGROUP = ROWS_PER_TILE // 8

NACC = 16 * 632
NDEG = 16 * 640

_BR = 1000


def _stats_body(x_ref, o_ref):
    i = pl.program_id(0)
    xb = x_ref[...]
    s = jnp.sum(xb, axis=0, keepdims=True)
    s2 = jnp.sum(xb * xb, axis=0, keepdims=True)

    @pl.when(i == 0)
    def _():
        o_ref[...] = jnp.zeros_like(o_ref)

    o_ref[0:1, :] += s
    o_ref[1:2, :] += s2

    @pl.when(i == pl.num_programs(0) - 1)
    def _():
        mean = o_ref[0:1, :] / N
        ms = o_ref[1:2, :] / N
        var = ms - mean * mean
        o_ref[0:1, :] = mean
        o_ref[1:2, :] = lax.rsqrt(var + 1e-5)


def _stats(x):
    return pl.pallas_call(
        _stats_body,
        grid=(N // _BR,),
        in_specs=[pl.BlockSpec((_BR, D), lambda i: (i, 0))],
        out_specs=pl.BlockSpec((8, D), lambda i: (0, 0)),
        out_shape=jax.ShapeDtypeStruct((8, D), jnp.float32),
    )(x)


def _mm1_body(x_ref, st_ref, g_ref, bta_ref, w_ref, up_ref, dn_ref, bs_ref):
    mean = st_ref[0:1, :]
    rstd = st_ref[1:2, :]
    xn = (x_ref[...] - mean) * (rstd * g_ref[...]) + bta_ref[...]
    y = jnp.dot(xn, w_ref[...], preferred_element_type=jnp.float32)
    up_ref[...] = y[:, :H]
    dn_ref[...] = y[:, H:2 * H]
    bs_ref[...] = y[:, 2 * H:]


def _mm1(x, stats, gamma, beta, w1cat):
    o = jax.ShapeDtypeStruct((N, H), jnp.float32)
    return pl.pallas_call(
        _mm1_body,
        grid=(N // _BR,),
        in_specs=[
            pl.BlockSpec((_BR, D), lambda i: (i, 0)),
            pl.BlockSpec((8, D), lambda i: (0, 0)),
            pl.BlockSpec((1, D), lambda i: (0, 0)),
            pl.BlockSpec((1, D), lambda i: (0, 0)),
            pl.BlockSpec((D, 3 * H), lambda i: (0, 0)),
        ],
        out_specs=[pl.BlockSpec((_BR, H), lambda i: (i, 0))] * 3,
        out_shape=[o, o, o],
    )(x, stats, gamma, beta, w1cat)


def _norm_cat(su_ref, sd_ref, bs_ref, du_ref, dv_ref):
    iu = 1.0 / jnp.maximum(du_ref[...], 1.0)
    iv = 1.0 / jnp.maximum(dv_ref[...], 1.0)
    cat = jnp.concatenate([su_ref[...] * iu, sd_ref[...] * iv, bs_ref[...]],
                          axis=1)
    ss = jnp.sum(cat * cat, axis=1, keepdims=True)
    l2 = jnp.maximum(jnp.sqrt(ss), 1e-12)
    cat = cat / l2
    return jnp.where(cat >= 0, cat, 0.1 * cat)


def _fused_body(su_ref, sd_ref, bs_ref, du_ref, dv_ref, w_ref,
                up_ref, dn_ref, bo_ref):
    h = _norm_cat(su_ref, sd_ref, bs_ref, du_ref, dv_ref)
    y = jnp.dot(h, w_ref[...], preferred_element_type=jnp.float32)
    up_ref[...] = y[:, :H]
    dn_ref[...] = y[:, H:2 * H]
    bo_ref[...] = y[:, 2 * H:]


def _fused(s_up, s_dn, bias, du, dv, wcat):
    o = jax.ShapeDtypeStruct((N, H), jnp.float32)
    blk = pl.BlockSpec((_BR, H), lambda i: (i, 0))
    dblk = pl.BlockSpec((_BR, 1), lambda i: (i, 0))
    return pl.pallas_call(
        _fused_body,
        grid=(N // _BR,),
        in_specs=[blk, blk, blk, dblk, dblk,
                  pl.BlockSpec((3 * H, 3 * H), lambda i: (0, 0))],
        out_specs=[blk] * 3,
        out_shape=[o, o, o],
    )(s_up, s_dn, bias, du, dv, wcat)


def _post3_body(su_ref, sd_ref, bs_ref, du_ref, dv_ref, h_ref):
    h_ref[...] = _norm_cat(su_ref, sd_ref, bs_ref, du_ref, dv_ref)


def _post3(s_up, s_dn, bias, du, dv):
    blk = pl.BlockSpec((_BR, H), lambda i: (i, 0))
    dblk = pl.BlockSpec((_BR, 1), lambda i: (i, 0))
    return pl.pallas_call(
        _post3_body,
        grid=(N // _BR,),
        in_specs=[blk, blk, blk, dblk, dblk],
        out_specs=pl.BlockSpec((_BR, 3 * H), lambda i: (i, 0)),
        out_shape=jax.ShapeDtypeStruct((N, 3 * H), jnp.float32),
    )(s_up, s_dn, bias, du, dv)


def _decode_body(a_ref, b_ref, p1_ref, p2_ref, o_ref):
    u = jnp.dot(a_ref[...], p1_ref[...], preferred_element_type=jnp.float32)
    v = jnp.dot(u, p2_ref[...], preferred_element_type=jnp.float32)
    w = jnp.dot(b_ref[...], p1_ref[...], preferred_element_type=jnp.float32)
    o_ref[...] = jnp.sum(v * w, axis=1, keepdims=True)


def _decode(a, b, p1, p2):
    bb = 256
    return pl.pallas_call(
        _decode_body,
        grid=(B // bb,),
        in_specs=[
            pl.BlockSpec((bb, 3 * H), lambda i: (i, 0)),
            pl.BlockSpec((bb, 3 * H), lambda i: (i, 0)),
            pl.BlockSpec((3 * H, 128), lambda i: (0, 0)),
            pl.BlockSpec((128, 128), lambda i: (0, 0)),
        ],
        out_specs=pl.BlockSpec((bb, 1), lambda i: (i, 0)),
        out_shape=jax.ShapeDtypeStruct((B, 1), jnp.float32),
    )(a, b, p1, p2)


_MESH = plsc.VectorSubcoreMesh(core_axis_name="c", subcore_axis_name="s")


def _agg_kernel_body(tu, td, su, du, sv, dv, ou, od,
                     acc, ia_s, ia_d, ib_s, ib_d, buf0, buf1,
                     gs0, gs1, ss0, ss1, isem):
    cid = lax.axis_index("c")
    sid = lax.axis_index("s")
    bufs = (buf0, buf1)
    gsems = (gs0, gs1)
    ssems = (ss0, ss1)

    zv = jnp.zeros((16,), jnp.float32)

    def zrow(r, carry):
        for k in range(8):
            buf0[r, pl.ds(k * 16, 16)] = zv
        return carry

    lax.fori_loop(0, 128, zrow, 0)
    off = sid * 632
    for c in range(4):
        pltpu.sync_copy(buf0, acc.at[pl.ds(off + c * 128, 128)])
    pltpu.sync_copy(buf0.at[pl.ds(0, 120)], acc.at[pl.ds(off + 512, 120)])
    plsc.subcore_barrier()

    def run_dir(tab, src2d, dst2d):
        base = sid * ROWS_PER_TILE
        ngrp = ROWS_PER_TILE // 16

        pltpu.sync_copy(src2d.at[pl.ds(base, 16)], ia_s)
        pltpu.sync_copy(dst2d.at[pl.ds(base, 16)], ia_d)

        def group(g, s_s, s_d, t_s, t_d):
            nxt = jnp.where(g + 1 < ngrp, base + (g + 1) * 16, base)
            p1 = pltpu.async_copy(src2d.at[pl.ds(nxt, 16)], t_s, isem)
            p2 = pltpu.async_copy(dst2d.at[pl.ds(nxt, 16)], t_d, isem)
            dg = [None, None]
            ds = [None, None]
            dg[0] = pltpu.async_copy(tab.at[s_s.at[0]], bufs[0], gsems[0])
            for j in range(16):
                b = j & 1
                dg[b].wait()
                if j < 15:
                    if j >= 1:
                        ds[1 - b].wait()
                    dg[1 - b] = pltpu.async_copy(tab.at[s_s.at[j + 1]],
                                                 bufs[1 - b], gsems[1 - b])
                ds[b] = pltpu.async_copy(bufs[b], acc.at[s_d.at[j]],
                                         ssems[b], add=True)
            ds[0].wait()
            ds[1].wait()
            p1.wait()
            p2.wait()

        def pair_body(k, carry):
            group(2 * k, ia_s, ia_d, ib_s, ib_d)
            group(2 * k + 1, ib_s, ib_d, ia_s, ia_d)
            return carry

        lax.fori_loop(0, ngrp // 2, pair_body, 0)

    @pl.when(cid == 0)
    def _():
        run_dir(tu, su, du)

    @pl.when(cid == 1)
    def _():
        run_dir(td, sv, dv)

    plsc.subcore_barrier()

    def wb(out_ref):
        def chunk(o2, nrow):
            pltpu.sync_copy(acc.at[pl.ds(off + o2, nrow)],
                            buf0.at[pl.ds(0, nrow)])
            pltpu.sync_copy(buf0.at[pl.ds(0, nrow)],
                            out_ref.at[pl.ds(off + o2, nrow)])

        @pl.when(sid < 15)
        def _():
            for c in range(4):
                chunk(c * 128, 128)
            chunk(512, 120)

        @pl.when(sid == 15)
        def _():
            for c in range(4):
                chunk(c * 128, 128)
            chunk(512, 8)

    @pl.when(cid == 0)
    def _():
        wb(ou)

    @pl.when(cid == 1)
    def _():
        wb(od)


def _agg(tab_up, tab_dn, src_up, dst_up, src_dn, dst_dn):
    o = jax.ShapeDtypeStruct((N, H), jnp.float32)
    f = pl.kernel(
        _agg_kernel_body,
        out_type=[o, o],
        mesh=_MESH,
        scratch_types=[
            pltpu.VMEM_SHARED((NACC, H), jnp.float32),
            pltpu.VMEM((16, 128), jnp.int32),
            pltpu.VMEM((16, 128), jnp.int32),
            pltpu.VMEM((16, 128), jnp.int32),
            pltpu.VMEM((16, 128), jnp.int32),
            pltpu.VMEM((128, H), jnp.float32),
            pltpu.VMEM((128, H), jnp.float32),
            pltpu.SemaphoreType.DMA,
            pltpu.SemaphoreType.DMA,
            pltpu.SemaphoreType.DMA,
            pltpu.SemaphoreType.DMA,
            pltpu.SemaphoreType.DMA,
        ],
    )
    return f(tab_up, tab_dn, src_up, dst_up, src_dn, dst_dn)


def _deg_kernel_body(du2d, dv2d, ou, od, dacc, didx, ones, zstage, sem):
    del sem
    cid = lax.axis_index("c")
    sid = lax.axis_index("s")

    ov = jnp.full((16,), 1.0, jnp.float32)
    for k in range(8):
        ones[pl.ds(k * 16, 16)] = ov
    zv = jnp.zeros((16,), jnp.float32)

    def zrow(r, carry):
        zstage[pl.ds(r * 16, 16)] = zv
        return carry

    lax.fori_loop(0, 40, zrow, 0)
    off = sid * 640
    pltpu.sync_copy(zstage, dacc.at[pl.ds(off, 640)])
    plsc.subcore_barrier()

    def run_dir(dst2d):
        base_row = sid * ROWS_PER_TILE

        def grp(g, carry):
            r0 = base_row + g * 8
            pltpu.sync_copy(dst2d.at[pl.ds(r0, 8)], didx)
            for j in range(8):
                pltpu.sync_copy(ones, dacc.at[didx.at[j]], add=True)
            return carry

        lax.fori_loop(0, NGROUP, grp, 0)

    @pl.when(cid == 0)
    def _():
        run_dir(du2d)

    @pl.when(cid == 1)
    def _():
        run_dir(dv2d)

    plsc.subcore_barrier()

    def wb(out_ref):
        @pl.when(sid < 15)
        def _():
            pltpu.sync_copy(dacc.at[pl.ds(off, 640)], zstage)
            pltpu.sync_copy(zstage, out_ref.at[pl.ds(off, 640)])

        @pl.when(sid == 15)
        def _():
            pltpu.sync_copy(dacc.at[pl.ds(off, 400)], zstage.at[pl.ds(0, 400)])
            pltpu.sync_copy(zstage.at[pl.ds(0, 400)],
                            out_ref.at[pl.ds(off, 400)])

    @pl.when(cid == 0)
    def _():
        wb(ou)

    @pl.when(cid == 1)
    def _():
        wb(od)


def _deg(dst_up, dst_dn):
    o = jax.ShapeDtypeStruct((N,), jnp.float32)
    f = pl.kernel(
        _deg_kernel_body,
        out_type=[o, o],
        mesh=_MESH,
        scratch_types=[
            pltpu.VMEM_SHARED((NDEG,), jnp.float32),
            pltpu.VMEM((8, 128), jnp.int32),
            pltpu.VMEM((128,), jnp.float32),
            pltpu.VMEM((640,), jnp.float32),
            pltpu.SemaphoreType.DMA,
        ],
    )
    return f(dst_up, dst_dn)


def _pair_kernel_body(h3, ia, ib, oa, ob, idxb, buf, sem):
    cid = lax.axis_index("c")
    sid = lax.axis_index("s")
    wid = sid * 2 + cid
    base = wid * 32

    pltpu.sync_copy(ia.at[pl.ds(base, 32)], idxb)
    pltpu.async_copy(h3.at[idxb], buf, sem).wait()
    pltpu.sync_copy(buf, oa.at[pl.ds(base, 32)])

    pltpu.sync_copy(ib.at[pl.ds(base, 32)], idxb)
    pltpu.async_copy(h3.at[idxb], buf, sem).wait()
    pltpu.sync_copy(buf, ob.at[pl.ds(base, 32)])


def _pair(h3, ia, ib):
    o = jax.ShapeDtypeStruct((B, 3 * H), jnp.float32)
    f = pl.kernel(
        _pair_kernel_body,
        out_type=[o, o],
        mesh=_MESH,
        scratch_types=[
            pltpu.VMEM((32,), jnp.int32),
            pltpu.VMEM((32, 3 * H), jnp.float32),
            pltpu.SemaphoreType.DMA,
        ],
    )
    return f(h3, ia, ib)


def kernel(x, edge_index, drug_index, bn_gamma, bn_beta,
           W_up1, W_down1, W_bias1,
           W_up2, W_down2, W_bias2,
           W_up3, W_down3, W_bias3,
           P1, P2):
    row = edge_index[0]
    col = edge_index[1]
    npad = EPAD - E
    pad_g = jnp.zeros((npad,), jnp.int32)
    pad_s = jnp.full((npad,), N, jnp.int32)
    shape2d = (EPAD // 128, 128)
    src_up = jnp.concatenate([row, pad_g]).reshape(shape2d)
    dst_up = jnp.concatenate([col, pad_s]).reshape(shape2d)
    src_dn = jnp.concatenate([col, pad_g]).reshape(shape2d)
    dst_dn = jnp.concatenate([row, pad_s]).reshape(shape2d)

    stats = _stats(x)
    w1cat = jnp.concatenate([W_up1, W_down1, W_bias1], axis=1)
    up1, dn1, b1 = _mm1(x, stats, bn_gamma.reshape(1, D),
                        bn_beta.reshape(1, D), w1cat)

    du, dv = _deg(dst_up, dst_dn)
    du = du.reshape(N, 1)
    dv = dv.reshape(N, 1)

    s_up, s_dn = _agg(up1, dn1, src_up, dst_up, src_dn, dst_dn)

    w2cat = jnp.concatenate([W_up2, W_down2, W_bias2], axis=1)
    up2, dn2, b2 = _fused(s_up, s_dn, b1, du, dv, w2cat)
    s_up, s_dn = _agg(up2, dn2, src_up, dst_up, src_dn, dst_dn)

    w3cat = jnp.concatenate([W_up3, W_down3, W_bias3], axis=1)
    up3, dn3, b3 = _fused(s_up, s_dn, b2, du, dv, w3cat)
    s_up, s_dn = _agg(up3, dn3, src_up, dst_up, src_dn, dst_dn)

    h3 = _post3(s_up, s_dn, b3, du, dv)

    ia = (drug_index[:, 0] - 1).astype(jnp.int32)
    ib = (drug_index[:, 1] - 1).astype(jnp.int32)
    a, b = _pair(h3, ia, ib)
    return _decode(a, b, P1, P2)
